# baseline (speedup 1.0000x reference)
import functools

import jax
import jax.numpy as jnp
from jax.experimental import pallas as pl

_N, _D = 16384, 50
_P = _D // 2
_ROWS = 1024
_KA = 128


def _body(xa_ref, m_ref, W1_ref, W2_ref, W3_ref, b3_ref,
          rW1_ref, rb1_ref, rW2_ref, rb2_ref, rW3_ref, rb3_ref,
          mu_ref, sig_ref):
    xa = xa_ref[:]
    W2 = W2_ref[:]
    s2 = jnp.zeros((_ROWS, 128), jnp.float32)
    for j in range(_P):
        W1j = W1_ref[_KA * j:_KA * (j + 1), :]
        p1 = jnp.dot(xa, W1j, preferred_element_type=jnp.float32)
        g = jnp.maximum(p1, 0.0).astype(jnp.bfloat16)
        p2 = jnp.dot(g, W2, preferred_element_type=jnp.float32)
        s2 = s2 + jnp.maximum(p2, 0.0)

    cnt = jnp.sum(m_ref[:], axis=1, keepdims=True)
    pooled = (jnp.dot(s2, W3_ref[:], preferred_element_type=jnp.float32)
              + cnt * b3_ref[:])
    r = jnp.maximum(
        jnp.dot(pooled, rW1_ref[:], preferred_element_type=jnp.float32)
        + rb1_ref[:], 0.0)
    r = jnp.maximum(
        jnp.dot(r, rW2_ref[:], preferred_element_type=jnp.float32)
        + rb2_ref[:], 0.0)
    g = (jnp.dot(r, rW3_ref[:], preferred_element_type=jnp.float32)
         + rb3_ref[:])
    mu_ref[:] = g[:, :64]
    sig_ref[:] = jnp.logaddexp(g[:, 64:], 0.0)


@functools.partial(jax.jit, static_argnames=("interpret",))
def _run(xa, maskf, W1s, W2a, W3s, b3, rW1, rb1, rW2, rb2, rW3, rb3,
         interpret=False):
    grid = (_N // _ROWS,)

    def rep(shape):
        return pl.BlockSpec(shape, lambda i: tuple(0 for _ in shape))

    mu, sig = pl.pallas_call(
        _body,
        grid=grid,
        in_specs=[
            pl.BlockSpec((_ROWS, _KA), lambda i: (i, 0)),
            pl.BlockSpec((_ROWS, _D), lambda i: (i, 0)),
            rep((_KA * _P, 130)), rep((130, 128)),
            rep((128, 64)), rep((1, 64)),
            rep((64, 64)), rep((1, 64)),
            rep((64, 64)), rep((1, 64)),
            rep((64, 128)), rep((1, 128)),
        ],
        out_specs=[pl.BlockSpec((_ROWS, 64), lambda i: (i, 0)),
                   pl.BlockSpec((_ROWS, 64), lambda i: (i, 0))],
        out_shape=[
            jax.ShapeDtypeStruct((_N, 64), jnp.float32),
            jax.ShapeDtypeStruct((_N, 64), jnp.float32),
        ],
        interpret=interpret,
    )(xa, maskf, W1s, W2a, W3s, b3, rW1, rb1, rW2, rb2, rW3, rb3)
    return mu, sig


def kernel(x, mask, hW1, hb1, hW2, hb2, hW3, hb3,
           rW1, rb1, rW2, rb2, rW3, rb3):
    maskf = mask.astype(jnp.float32)
    xa = jnp.stack([x * maskf, maskf], axis=2).reshape(_N, 2 * _D)
    xa = jnp.pad(xa, ((0, 0), (0, _KA - 2 * _D))).astype(jnp.bfloat16)

    dim_ids = jnp.arange(_D, dtype=jnp.float32)[:, None]
    B = dim_ids * hW1[1:2, :] + hb1[None, :]
    w0 = hW1[0, :]

    W1s = jnp.zeros((_P, _KA, 130), jnp.float32)
    pj = jnp.arange(_P)
    W1s = W1s.at[pj, 4 * pj + 0, 0:64].set(w0[None, :])
    W1s = W1s.at[pj, 4 * pj + 1, 0:64].set(B[2 * pj])
    W1s = W1s.at[pj, 4 * pj + 1, 128].set(1.0)
    W1s = W1s.at[pj, 4 * pj + 2, 64:128].set(w0[None, :])
    W1s = W1s.at[pj, 4 * pj + 3, 64:128].set(B[2 * pj + 1])
    W1s = W1s.at[pj, 4 * pj + 3, 129].set(1.0)
    W1s = W1s.reshape(_P * _KA, 130).astype(jnp.bfloat16)

    z = jnp.zeros((64, 64), jnp.float32)
    W2a = jnp.block([[hW2, z], [z, hW2],
                     [hb2[None, :], jnp.zeros((1, 64), jnp.float32)],
                     [jnp.zeros((1, 64), jnp.float32), hb2[None, :]]])
    W2a = W2a.astype(jnp.bfloat16)

    W3s = jnp.concatenate([hW3, hW3], axis=0)

    return _run(xa, maskf, W1s, W2a, W3s, hb3[None, :],
                rW1, rb1[None, :], rW2, rb2[None, :], rW3, rb3[None, :])


# speedup vs baseline: 3.5647x; 3.5647x over previous
import functools

import jax
import jax.numpy as jnp
from jax.experimental import pallas as pl

_N, _D = 16384, 50
_P = _D // 2
_ROWS = 1024
_KA = 128


def _body(xa_ref, m_ref, W1_ref, W2_ref, W3_ref, b3_ref,
          rW1_ref, rb1_ref, rW2_ref, rb2_ref, rW3_ref, rb3_ref,
          mu_ref, sig_ref):
    xa = xa_ref[:]
    W2 = W2_ref[:]
    s2 = jnp.zeros((_ROWS, 128), jnp.float32)
    for j in range(_P):
        W1j = W1_ref[_KA * j:_KA * (j + 1), :]
        p1 = jnp.dot(xa, W1j, preferred_element_type=jnp.float32)
        g = jnp.maximum(p1, 0.0).astype(jnp.bfloat16)
        p2 = jnp.dot(g, W2, preferred_element_type=jnp.float32)
        s2 = s2 + jnp.maximum(p2, 0.0)

    cnt = jnp.sum(m_ref[:], axis=1, keepdims=True)
    pooled = (jnp.dot(s2, W3_ref[:], preferred_element_type=jnp.float32)
              + cnt * b3_ref[:])
    r = jnp.maximum(
        jnp.dot(pooled, rW1_ref[:], preferred_element_type=jnp.float32)
        + rb1_ref[:], 0.0)
    r = jnp.maximum(
        jnp.dot(r, rW2_ref[:], preferred_element_type=jnp.float32)
        + rb2_ref[:], 0.0)
    g = (jnp.dot(r, rW3_ref[:], preferred_element_type=jnp.float32)
         + rb3_ref[:])
    mu_ref[:] = g[:, :64]
    sig_ref[:] = jnp.logaddexp(g[:, 64:], 0.0)


@functools.partial(jax.jit, static_argnames=("interpret",))
def _run(xa, maskf, W1s, W2a, W3s, b3, rW1, rb1, rW2, rb2, rW3, rb3,
         interpret=False):
    grid = (_N // _ROWS,)

    def rep(shape):
        return pl.BlockSpec(shape, lambda i: tuple(0 for _ in shape))

    mu, sig = pl.pallas_call(
        _body,
        grid=grid,
        in_specs=[
            pl.BlockSpec((_ROWS, _KA), lambda i: (i, 0)),
            pl.BlockSpec((_ROWS, _D), lambda i: (i, 0)),
            rep((_KA * _P, 130)), rep((130, 128)),
            rep((128, 64)), rep((1, 64)),
            rep((64, 64)), rep((1, 64)),
            rep((64, 64)), rep((1, 64)),
            rep((64, 128)), rep((1, 128)),
        ],
        out_specs=[pl.BlockSpec((_ROWS, 64), lambda i: (i, 0)),
                   pl.BlockSpec((_ROWS, 64), lambda i: (i, 0))],
        out_shape=[
            jax.ShapeDtypeStruct((_N, 64), jnp.float32),
            jax.ShapeDtypeStruct((_N, 64), jnp.float32),
        ],
        interpret=interpret,
    )(xa, maskf, W1s, W2a, W3s, b3, rW1, rb1, rW2, rb2, rW3, rb3)
    return mu, sig


def kernel(x, mask, hW1, hb1, hW2, hb2, hW3, hb3,
           rW1, rb1, rW2, rb2, rW3, rb3):
    maskf = mask.astype(jnp.float32)
    xa = jnp.stack([x * maskf, maskf], axis=2).reshape(_N, 2 * _D)
    xa = jnp.pad(xa, ((0, 0), (0, _KA - 2 * _D))).astype(jnp.bfloat16)

    dim_ids = jnp.arange(_D, dtype=jnp.float32)[:, None]
    B = dim_ids * hW1[1:2, :] + hb1[None, :]
    w0 = hW1[0, :]

    z64 = jnp.zeros((64,), jnp.float32)
    zP64 = jnp.zeros((_P, 64), jnp.float32)
    oP = jnp.ones((_P, 1), jnp.float32)
    zP = jnp.zeros((_P, 1), jnp.float32)
    row0 = jnp.concatenate([w0, z64, jnp.zeros((2,), jnp.float32)])
    row2 = jnp.concatenate([z64, w0, jnp.zeros((2,), jnp.float32)])
    row1 = jnp.concatenate([B[0::2], zP64, oP, zP], axis=1)
    row3 = jnp.concatenate([zP64, B[1::2], zP, oP], axis=1)
    r_iota = jnp.arange(_KA)[None, :, None]
    base = 4 * jnp.arange(_P)[:, None, None]
    W1s = ((r_iota == base) * row0[None, None, :]
           + (r_iota == base + 1) * row1[:, None, :]
           + (r_iota == base + 2) * row2[None, None, :]
           + (r_iota == base + 3) * row3[:, None, :])
    W1s = W1s.reshape(_P * _KA, 130).astype(jnp.bfloat16)

    z = jnp.zeros((64, 64), jnp.float32)
    W2a = jnp.block([[hW2, z], [z, hW2],
                     [hb2[None, :], jnp.zeros((1, 64), jnp.float32)],
                     [jnp.zeros((1, 64), jnp.float32), hb2[None, :]]])
    W2a = W2a.astype(jnp.bfloat16)

    W3s = jnp.concatenate([hW3, hW3], axis=0)

    return _run(xa, maskf, W1s, W2a, W3s, hb3[None, :],
                rW1, rb1[None, :], rW2, rb2[None, :], rW3, rb3[None, :])


# speedup vs baseline: 4.1031x; 1.1510x over previous
import functools

import jax
import jax.numpy as jnp
from jax.experimental import pallas as pl

_N, _D = 16384, 50
_P = _D // 2
_ROWS = 1024
_KA = 128


def _body(xa_ref, W1_ref, W2_ref, W3_ref, b3_ref,
          rW1_ref, rb1_ref, rW2_ref, rb2_ref, rW3_ref, rb3_ref,
          mu_ref, sig_ref):
    xa = xa_ref[:]
    W2 = W2_ref[:]
    s2 = jnp.zeros((_ROWS, 128), jnp.float32)
    for j in range(_P):
        W1j = W1_ref[_KA * j:_KA * (j + 1), :]
        p1 = jnp.dot(xa, W1j, preferred_element_type=jnp.float32)
        g = jnp.maximum(p1, 0.0).astype(jnp.bfloat16)
        p2 = jnp.dot(g, W2, preferred_element_type=jnp.float32)
        s2 = s2 + jnp.maximum(p2, 0.0)

    cnt = jnp.sum(xa[:, _D:2 * _D].astype(jnp.float32), axis=1,
                  keepdims=True)
    pooled = (jnp.dot(s2, W3_ref[:], preferred_element_type=jnp.float32)
              + cnt * b3_ref[:])
    r = jnp.maximum(
        jnp.dot(pooled, rW1_ref[:], preferred_element_type=jnp.float32)
        + rb1_ref[:], 0.0)
    r = jnp.maximum(
        jnp.dot(r, rW2_ref[:], preferred_element_type=jnp.float32)
        + rb2_ref[:], 0.0)
    g = (jnp.dot(r, rW3_ref[:], preferred_element_type=jnp.float32)
         + rb3_ref[:])
    mu_ref[:] = g[:, :64]
    sig_ref[:] = jnp.logaddexp(g[:, 64:], 0.0)


@functools.partial(jax.jit, static_argnames=("interpret",))
def _run(xa, W1s, W2a, W3s, b3, rW1, rb1, rW2, rb2, rW3, rb3,
         interpret=False):
    grid = (_N // _ROWS,)

    def rep(shape):
        return pl.BlockSpec(shape, lambda i: tuple(0 for _ in shape))

    mu, sig = pl.pallas_call(
        _body,
        grid=grid,
        in_specs=[
            pl.BlockSpec((_ROWS, _KA), lambda i: (i, 0)),
            rep((_KA * _P, 130)), rep((130, 128)),
            rep((128, 64)), rep((1, 64)),
            rep((64, 64)), rep((1, 64)),
            rep((64, 64)), rep((1, 64)),
            rep((64, 128)), rep((1, 128)),
        ],
        out_specs=[pl.BlockSpec((_ROWS, 64), lambda i: (i, 0)),
                   pl.BlockSpec((_ROWS, 64), lambda i: (i, 0))],
        out_shape=[
            jax.ShapeDtypeStruct((_N, 64), jnp.float32),
            jax.ShapeDtypeStruct((_N, 64), jnp.float32),
        ],
        interpret=interpret,
    )(xa, W1s, W2a, W3s, b3, rW1, rb1, rW2, rb2, rW3, rb3)
    return mu, sig


def kernel(x, mask, hW1, hb1, hW2, hb2, hW3, hb3,
           rW1, rb1, rW2, rb2, rW3, rb3):
    maskf = mask.astype(jnp.float32)
    xa = jnp.concatenate([x * maskf, maskf], axis=1)
    xa = jnp.pad(xa, ((0, 0), (0, _KA - 2 * _D))).astype(jnp.bfloat16)

    dim_ids = jnp.arange(_D, dtype=jnp.float32)[:, None]
    B = dim_ids * hW1[1:2, :] + hb1[None, :]
    w0 = hW1[0, :]

    z2 = jnp.zeros((2,), jnp.float32)
    z64 = jnp.zeros((64,), jnp.float32)
    zP64 = jnp.zeros((_P, 64), jnp.float32)
    oP = jnp.ones((_P, 1), jnp.float32)
    zP = jnp.zeros((_P, 1), jnp.float32)
    row_xL = jnp.concatenate([w0, z64, z2])
    row_xR = jnp.concatenate([z64, w0, z2])
    row_mL = jnp.concatenate([B[0::2], zP64, oP, zP], axis=1)
    row_mR = jnp.concatenate([zP64, B[1::2], zP, oP], axis=1)
    r_iota = jnp.arange(_KA)[None, :, None]
    base = 2 * jnp.arange(_P)[:, None, None]
    W1s = ((r_iota == base) * row_xL[None, None, :]
           + (r_iota == base + 1) * row_xR[None, None, :]
           + (r_iota == base + _D) * row_mL[:, None, :]
           + (r_iota == base + _D + 1) * row_mR[:, None, :])
    W1s = W1s.reshape(_P * _KA, 130).astype(jnp.bfloat16)

    z = jnp.zeros((64, 64), jnp.float32)
    W2a = jnp.block([[hW2, z], [z, hW2],
                     [hb2[None, :], jnp.zeros((1, 64), jnp.float32)],
                     [jnp.zeros((1, 64), jnp.float32), hb2[None, :]]])
    W2a = W2a.astype(jnp.bfloat16)

    W3s = jnp.concatenate([hW3, hW3], axis=0)

    return _run(xa, W1s, W2a, W3s, hb3[None, :],
                rW1, rb1[None, :], rW2, rb2[None, :], rW3, rb3[None, :])


# speedup vs baseline: 5.8121x; 1.4165x over previous
import functools

import jax
import jax.numpy as jnp
from jax.experimental import pallas as pl

_N, _D = 16384, 50
_P = _D // 2
_ROWS = 1024
_KA = 128


def _body(xa_ref, W1_ref, W2_ref, b2_ref, W3_ref, bc_ref,
          rW1_ref, rb1_ref, rW2_ref, rb2_ref, rW3_ref, rb3_ref,
          mu_ref, sig_ref):
    xa = xa_ref[:]
    W2 = W2_ref[:]
    b2 = b2_ref[:]
    s2 = jnp.zeros((_ROWS, 128), jnp.float32)
    for j in range(_P):
        W1j = W1_ref[_KA * j:_KA * (j + 1), :]
        p1 = jnp.dot(xa, W1j, preferred_element_type=jnp.float32)
        g = jnp.maximum(p1.astype(jnp.bfloat16), jnp.bfloat16(0.0))
        p2 = jnp.dot(g, W2, preferred_element_type=jnp.float32)
        s2 = s2 + jnp.maximum(p2 + b2, 0.0)

    cnt = jnp.sum(xa[:, _D:2 * _D].astype(jnp.float32), axis=1,
                  keepdims=True)
    pooled = (jnp.dot(s2, W3_ref[:], preferred_element_type=jnp.float32)
              + cnt * bc_ref[:])
    r = jnp.maximum(
        jnp.dot(pooled, rW1_ref[:], preferred_element_type=jnp.float32)
        + rb1_ref[:], 0.0)
    r = jnp.maximum(
        jnp.dot(r, rW2_ref[:], preferred_element_type=jnp.float32)
        + rb2_ref[:], 0.0)
    g = (jnp.dot(r, rW3_ref[:], preferred_element_type=jnp.float32)
         + rb3_ref[:])
    mu_ref[:] = g[:, :64]
    sig_ref[:] = jnp.logaddexp(g[:, 64:], 0.0)


@functools.partial(jax.jit, static_argnames=("interpret",))
def _run(xa, W1s, W2b, b2b, W3s, bc, rW1, rb1c, rW2, rb2, rW3, rb3,
         interpret=False):
    grid = (_N // _ROWS,)

    def rep(shape):
        return pl.BlockSpec(shape, lambda i: tuple(0 for _ in shape))

    mu, sig = pl.pallas_call(
        _body,
        grid=grid,
        in_specs=[
            pl.BlockSpec((_ROWS, _KA), lambda i: (i, 0)),
            rep((_KA * _P, 128)), rep((128, 128)), rep((1, 128)),
            rep((128, 64)), rep((1, 64)),
            rep((64, 64)), rep((1, 64)),
            rep((64, 64)), rep((1, 64)),
            rep((64, 128)), rep((1, 128)),
        ],
        out_specs=[pl.BlockSpec((_ROWS, 64), lambda i: (i, 0)),
                   pl.BlockSpec((_ROWS, 64), lambda i: (i, 0))],
        out_shape=[
            jax.ShapeDtypeStruct((_N, 64), jnp.float32),
            jax.ShapeDtypeStruct((_N, 64), jnp.float32),
        ],
        interpret=interpret,
    )(xa, W1s, W2b, b2b, W3s, bc, rW1, rb1c, rW2, rb2, rW3, rb3)
    return mu, sig


def kernel(x, mask, hW1, hb1, hW2, hb2, hW3, hb3,
           rW1, rb1, rW2, rb2, rW3, rb3):
    maskf = mask.astype(jnp.float32)
    xa = jnp.concatenate([x * maskf, maskf], axis=1)
    xa = jnp.pad(xa, ((0, 0), (0, _KA - 2 * _D))).astype(jnp.bfloat16)

    dim_ids = jnp.arange(_D, dtype=jnp.float32)[:, None]
    B = dim_ids * hW1[1:2, :] + hb1[None, :]
    w0 = hW1[0, :]

    z64 = jnp.zeros((64,), jnp.float32)
    zP64 = jnp.zeros((_P, 64), jnp.float32)
    row_xL = jnp.concatenate([w0, z64])
    row_xR = jnp.concatenate([z64, w0])
    row_mL = jnp.concatenate([B[0::2], zP64], axis=1)
    row_mR = jnp.concatenate([zP64, B[1::2]], axis=1)
    r_iota = jnp.arange(_KA)[None, :, None]
    base = 2 * jnp.arange(_P)[:, None, None]
    W1s = ((r_iota == base) * row_xL[None, None, :]
           + (r_iota == base + 1) * row_xR[None, None, :]
           + (r_iota == base + _D) * row_mL[:, None, :]
           + (r_iota == base + _D + 1) * row_mR[:, None, :])
    W1s = W1s.reshape(_P * _KA, _KA).astype(jnp.bfloat16)

    z = jnp.zeros((64, 64), jnp.float32)
    W2b = jnp.block([[hW2, z], [z, hW2]]).astype(jnp.bfloat16)
    b2b = jnp.concatenate([hb2, hb2])[None, :]

    W3s = jnp.concatenate([hW3, hW3], axis=0)

    q = jax.nn.relu(hb2) @ hW3
    bc = (hb3 + q)[None, :]
    rb1c = (rb1 - _D * (q @ rW1))[None, :]

    return _run(xa, W1s, W2b, b2b, W3s, bc,
                rW1, rb1c, rW2, rb2[None, :], rW3, rb3[None, :])


# speedup vs baseline: 6.0989x; 1.0493x over previous
import functools

import jax
import jax.numpy as jnp
from jax.experimental import pallas as pl

_N, _D = 16384, 50
_P = _D // 2
_ROWS = 2048
_KA = 128


def _body(xa_ref, W1_ref, W2_ref, b2_ref, W3_ref, bc_ref,
          rW1_ref, rb1_ref, rW2_ref, rb2_ref, rW3_ref, rb3_ref,
          mu_ref, sig_ref):
    xa = xa_ref[:]
    W2 = W2_ref[:]
    b2 = b2_ref[:]
    s2 = jnp.zeros((_ROWS, 128), jnp.float32)
    for j in range(_P):
        W1j = W1_ref[_KA * j:_KA * (j + 1), :]
        p1 = jnp.dot(xa, W1j, preferred_element_type=jnp.float32)
        g = jnp.maximum(p1.astype(jnp.bfloat16), jnp.bfloat16(0.0))
        p2 = jnp.dot(g, W2, preferred_element_type=jnp.float32)
        s2 = s2 + jnp.maximum(p2 + b2, 0.0)

    cnt = jnp.sum(xa[:, _D:2 * _D].astype(jnp.float32), axis=1,
                  keepdims=True)
    pooled = (jnp.dot(s2, W3_ref[:], preferred_element_type=jnp.float32)
              + cnt * bc_ref[:])
    r = jnp.maximum(
        jnp.dot(pooled, rW1_ref[:], preferred_element_type=jnp.float32)
        + rb1_ref[:], 0.0)
    r = jnp.maximum(
        jnp.dot(r, rW2_ref[:], preferred_element_type=jnp.float32)
        + rb2_ref[:], 0.0)
    g = (jnp.dot(r, rW3_ref[:], preferred_element_type=jnp.float32)
         + rb3_ref[:])
    mu_ref[:] = g[:, :64]
    sig_ref[:] = jnp.logaddexp(g[:, 64:], 0.0)


@functools.partial(jax.jit, static_argnames=("interpret",))
def _run(xa, W1s, W2b, b2b, W3s, bc, rW1, rb1c, rW2, rb2, rW3, rb3,
         interpret=False):
    grid = (_N // _ROWS,)

    def rep(shape):
        return pl.BlockSpec(shape, lambda i: tuple(0 for _ in shape))

    mu, sig = pl.pallas_call(
        _body,
        grid=grid,
        in_specs=[
            pl.BlockSpec((_ROWS, _KA), lambda i: (i, 0)),
            rep((_KA * _P, 128)), rep((128, 128)), rep((1, 128)),
            rep((128, 64)), rep((1, 64)),
            rep((64, 64)), rep((1, 64)),
            rep((64, 64)), rep((1, 64)),
            rep((64, 128)), rep((1, 128)),
        ],
        out_specs=[pl.BlockSpec((_ROWS, 64), lambda i: (i, 0)),
                   pl.BlockSpec((_ROWS, 64), lambda i: (i, 0))],
        out_shape=[
            jax.ShapeDtypeStruct((_N, 64), jnp.float32),
            jax.ShapeDtypeStruct((_N, 64), jnp.float32),
        ],
        interpret=interpret,
    )(xa, W1s, W2b, b2b, W3s, bc, rW1, rb1c, rW2, rb2, rW3, rb3)
    return mu, sig


def kernel(x, mask, hW1, hb1, hW2, hb2, hW3, hb3,
           rW1, rb1, rW2, rb2, rW3, rb3):
    maskf = mask.astype(jnp.float32)
    xa = jnp.concatenate([x * maskf, maskf], axis=1)
    xa = jnp.pad(xa, ((0, 0), (0, _KA - 2 * _D))).astype(jnp.bfloat16)

    dim_ids = jnp.arange(_D, dtype=jnp.float32)[:, None]
    B = dim_ids * hW1[1:2, :] + hb1[None, :]
    w0 = hW1[0, :]

    z64 = jnp.zeros((64,), jnp.float32)
    zP64 = jnp.zeros((_P, 64), jnp.float32)
    row_xL = jnp.concatenate([w0, z64])
    row_xR = jnp.concatenate([z64, w0])
    row_mL = jnp.concatenate([B[0::2], zP64], axis=1)
    row_mR = jnp.concatenate([zP64, B[1::2]], axis=1)
    r_iota = jnp.arange(_KA)[None, :, None]
    base = 2 * jnp.arange(_P)[:, None, None]
    W1s = ((r_iota == base) * row_xL[None, None, :]
           + (r_iota == base + 1) * row_xR[None, None, :]
           + (r_iota == base + _D) * row_mL[:, None, :]
           + (r_iota == base + _D + 1) * row_mR[:, None, :])
    W1s = W1s.reshape(_P * _KA, _KA).astype(jnp.bfloat16)

    z = jnp.zeros((64, 64), jnp.float32)
    W2b = jnp.block([[hW2, z], [z, hW2]]).astype(jnp.bfloat16)
    b2b = jnp.concatenate([hb2, hb2])[None, :]

    W3s = jnp.concatenate([hW3, hW3], axis=0)

    q = jax.nn.relu(hb2) @ hW3
    bc = (hb3 + q)[None, :]
    rb1c = (rb1 - _D * (q @ rW1))[None, :]

    return _run(xa, W1s, W2b, b2b, W3s, bc,
                rW1, rb1c, rW2, rb2[None, :], rW3, rb3[None, :])


# speedup vs baseline: 6.1137x; 1.0024x over previous
import functools

import jax
import jax.numpy as jnp
from jax.experimental import pallas as pl

_N, _D = 16384, 50
_P = _D // 2
_ROWS = 4096
_KA = 128


def _body(xa_ref, W1_ref, W2_ref, b2_ref, W3_ref, bc_ref,
          rW1_ref, rb1_ref, rW2_ref, rb2_ref, rW3_ref, rb3_ref,
          mu_ref, sig_ref):
    xa = xa_ref[:]
    W2 = W2_ref[:]
    b2 = b2_ref[:]
    s2 = jnp.zeros((_ROWS, 128), jnp.float32)
    for j in range(_P):
        W1j = W1_ref[_KA * j:_KA * (j + 1), :]
        p1 = jnp.dot(xa, W1j, preferred_element_type=jnp.float32)
        g = jnp.maximum(p1.astype(jnp.bfloat16), jnp.bfloat16(0.0))
        p2 = jnp.dot(g, W2, preferred_element_type=jnp.float32)
        s2 = s2 + jnp.maximum(p2 + b2, 0.0)

    cnt = jnp.sum(xa[:, _D:2 * _D].astype(jnp.float32), axis=1,
                  keepdims=True)
    pooled = (jnp.dot(s2, W3_ref[:], preferred_element_type=jnp.float32)
              + cnt * bc_ref[:])
    r = jnp.maximum(
        jnp.dot(pooled, rW1_ref[:], preferred_element_type=jnp.float32)
        + rb1_ref[:], 0.0)
    r = jnp.maximum(
        jnp.dot(r, rW2_ref[:], preferred_element_type=jnp.float32)
        + rb2_ref[:], 0.0)
    g = (jnp.dot(r, rW3_ref[:], preferred_element_type=jnp.float32)
         + rb3_ref[:])
    mu_ref[:] = g[:, :64]
    sig_ref[:] = jnp.logaddexp(g[:, 64:], 0.0)


@functools.partial(jax.jit, static_argnames=("interpret",))
def _run(xa, W1s, W2b, b2b, W3s, bc, rW1, rb1c, rW2, rb2, rW3, rb3,
         interpret=False):
    grid = (_N // _ROWS,)

    def rep(shape):
        return pl.BlockSpec(shape, lambda i: tuple(0 for _ in shape))

    mu, sig = pl.pallas_call(
        _body,
        grid=grid,
        in_specs=[
            pl.BlockSpec((_ROWS, _KA), lambda i: (i, 0)),
            rep((_KA * _P, 128)), rep((128, 128)), rep((1, 128)),
            rep((128, 64)), rep((1, 64)),
            rep((64, 64)), rep((1, 64)),
            rep((64, 64)), rep((1, 64)),
            rep((64, 128)), rep((1, 128)),
        ],
        out_specs=[pl.BlockSpec((_ROWS, 64), lambda i: (i, 0)),
                   pl.BlockSpec((_ROWS, 64), lambda i: (i, 0))],
        out_shape=[
            jax.ShapeDtypeStruct((_N, 64), jnp.float32),
            jax.ShapeDtypeStruct((_N, 64), jnp.float32),
        ],
        interpret=interpret,
    )(xa, W1s, W2b, b2b, W3s, bc, rW1, rb1c, rW2, rb2, rW3, rb3)
    return mu, sig


def kernel(x, mask, hW1, hb1, hW2, hb2, hW3, hb3,
           rW1, rb1, rW2, rb2, rW3, rb3):
    maskf = mask.astype(jnp.float32)
    xa = jnp.concatenate([x * maskf, maskf], axis=1)
    xa = jnp.pad(xa, ((0, 0), (0, _KA - 2 * _D))).astype(jnp.bfloat16)

    dim_ids = jnp.arange(_D, dtype=jnp.float32)[:, None]
    B = dim_ids * hW1[1:2, :] + hb1[None, :]
    w0 = hW1[0, :]

    z64 = jnp.zeros((64,), jnp.float32)
    zP64 = jnp.zeros((_P, 64), jnp.float32)
    row_xL = jnp.concatenate([w0, z64])
    row_xR = jnp.concatenate([z64, w0])
    row_mL = jnp.concatenate([B[0::2], zP64], axis=1)
    row_mR = jnp.concatenate([zP64, B[1::2]], axis=1)
    r_iota = jnp.arange(_KA)[None, :, None]
    base = 2 * jnp.arange(_P)[:, None, None]
    W1s = ((r_iota == base) * row_xL[None, None, :]
           + (r_iota == base + 1) * row_xR[None, None, :]
           + (r_iota == base + _D) * row_mL[:, None, :]
           + (r_iota == base + _D + 1) * row_mR[:, None, :])
    W1s = W1s.reshape(_P * _KA, _KA).astype(jnp.bfloat16)

    z = jnp.zeros((64, 64), jnp.float32)
    W2b = jnp.block([[hW2, z], [z, hW2]]).astype(jnp.bfloat16)
    b2b = jnp.concatenate([hb2, hb2])[None, :]

    W3s = jnp.concatenate([hW3, hW3], axis=0)

    q = jax.nn.relu(hb2) @ hW3
    bc = (hb3 + q)[None, :]
    rb1c = (rb1 - _D * (q @ rW1))[None, :]

    return _run(xa, W1s, W2b, b2b, W3s, bc,
                rW1, rb1c, rW2, rb2[None, :], rW3, rb3[None, :])
